# Initial kernel scaffold; baseline (speedup 1.0000x reference)
import jax
import jax.numpy as jnp
from jax.experimental import pallas as pl


def kernel(x, rW, rb, W1, b1, W2, b2):
    raise NotImplementedError("write your pallas kernel here")


# speedup vs baseline: 2.4822x; 2.4822x over previous
import functools

import jax
import jax.numpy as jnp
from jax.experimental import pallas as pl
from jax.experimental.pallas import tpu as pltpu

DIM = 768
HID = 768
E = 8
TT = 512


def _moe_body(x_ref, rW_ref, rb_ref, W1_ref, b1_ref, W2_ref, b2_ref,
              out_ref, w_scratch):
    e = pl.program_id(1)
    x = x_ref[...]

    @pl.when(e == 0)
    def _router():
        logits = jnp.dot(x, rW_ref[...], preferred_element_type=jnp.float32)
        logits = logits + rb_ref[0]
        m = jnp.max(logits, axis=-1, keepdims=True)
        p = jnp.exp(logits - m)
        w_scratch[...] = p / jnp.sum(p, axis=-1, keepdims=True)

    h = jnp.dot(x, W1_ref[0], preferred_element_type=jnp.float32) + b1_ref[0, 0]
    h = 0.5 * h * (1.0 + jax.lax.erf(h * 0.7071067811865476))
    eo = jnp.dot(h, W2_ref[0], preferred_element_type=jnp.float32) + b2_ref[0, 0]

    lane = jax.lax.broadcasted_iota(jnp.int32, (TT, E), 1)
    w_e = jnp.sum(jnp.where(lane == e, w_scratch[...], 0.0), axis=-1,
                  keepdims=True)
    contrib = w_e * eo

    @pl.when(e == 0)
    def _init():
        out_ref[...] = contrib

    @pl.when(e != 0)
    def _acc():
        out_ref[...] += contrib


@functools.partial(jax.jit, static_argnames=())
def kernel(x, rW, rb, W1, b1, W2, b2):
    B, T, D = x.shape
    x2 = x.reshape(T, D)
    grid = (T // TT, E)
    out = pl.pallas_call(
        _moe_body,
        grid=grid,
        in_specs=[
            pl.BlockSpec((TT, DIM), lambda t, e: (t, 0)),
            pl.BlockSpec((DIM, E), lambda t, e: (0, 0)),
            pl.BlockSpec((1, E), lambda t, e: (0, 0)),
            pl.BlockSpec((1, DIM, HID), lambda t, e: (e, 0, 0)),
            pl.BlockSpec((1, 1, HID), lambda t, e: (e, 0, 0)),
            pl.BlockSpec((1, HID, DIM), lambda t, e: (e, 0, 0)),
            pl.BlockSpec((1, 1, DIM), lambda t, e: (e, 0, 0)),
        ],
        out_specs=pl.BlockSpec((TT, DIM), lambda t, e: (t, 0)),
        out_shape=jax.ShapeDtypeStruct((T, DIM), jnp.float32),
        scratch_shapes=[pltpu.VMEM((TT, E), jnp.float32)],
        compiler_params=pltpu.CompilerParams(
            dimension_semantics=("parallel", "arbitrary"),
        ),
    )(x2, rW, rb.reshape(1, E), W1, b1.reshape(E, 1, HID), W2,
      b2.reshape(E, 1, DIM))
    return out.reshape(B, T, D)


# speedup vs baseline: 3.0006x; 1.2088x over previous
import jax
import jax.numpy as jnp
from jax.experimental import pallas as pl
from jax.experimental.pallas import tpu as pltpu

DIM = 768
HID = 768
E = 8
T = 2048


def _moe_body(x_ref, rW_ref, rb_ref, W1_ref, b1_ref, W2_ref, b2_ref,
              out_ref, w_scratch):
    e = pl.program_id(0)
    x = x_ref[...]

    @pl.when(e == 0)
    def _router():
        logits = jnp.dot(x, rW_ref[...], preferred_element_type=jnp.float32)
        logits = logits + rb_ref[0]
        m = jnp.max(logits, axis=-1, keepdims=True)
        p = jnp.exp(logits - m)
        w_scratch[...] = p / jnp.sum(p, axis=-1, keepdims=True)

    h = jnp.dot(x, W1_ref[0], preferred_element_type=jnp.float32) + b1_ref[0, 0]
    h = 0.5 * h * (1.0 + jax.lax.erf(h * 0.7071067811865476))
    eo = jnp.dot(h, W2_ref[0], preferred_element_type=jnp.float32) + b2_ref[0, 0]

    lane = jax.lax.broadcasted_iota(jnp.int32, (T, E), 1)
    w_e = jnp.sum(jnp.where(lane == e, w_scratch[...], 0.0), axis=-1,
                  keepdims=True)
    contrib = w_e * eo

    @pl.when(e == 0)
    def _init():
        out_ref[...] = contrib

    @pl.when(e != 0)
    def _acc():
        out_ref[...] += contrib


def kernel(x, rW, rb, W1, b1, W2, b2):
    B, Tx, D = x.shape
    x2 = x.reshape(Tx, D)
    out = pl.pallas_call(
        _moe_body,
        grid=(E,),
        in_specs=[
            pl.BlockSpec((T, DIM), lambda e: (0, 0)),
            pl.BlockSpec((DIM, E), lambda e: (0, 0)),
            pl.BlockSpec((1, E), lambda e: (0, 0)),
            pl.BlockSpec((1, DIM, HID), lambda e: (e, 0, 0)),
            pl.BlockSpec((1, 1, HID), lambda e: (e, 0, 0)),
            pl.BlockSpec((1, HID, DIM), lambda e: (e, 0, 0)),
            pl.BlockSpec((1, 1, DIM), lambda e: (e, 0, 0)),
        ],
        out_specs=pl.BlockSpec((T, DIM), lambda e: (0, 0)),
        out_shape=jax.ShapeDtypeStruct((Tx, DIM), jnp.float32),
        scratch_shapes=[pltpu.VMEM((T, E), jnp.float32)],
        compiler_params=pltpu.CompilerParams(
            dimension_semantics=("arbitrary",),
        ),
    )(x2, rW, rb.reshape(1, E), W1, b1.reshape(E, 1, HID), W2,
      b2.reshape(E, 1, DIM))
    return out.reshape(B, Tx, D)
